# Initial kernel scaffold; baseline (speedup 1.0000x reference)
import jax
import jax.numpy as jnp
from jax.experimental import pallas as pl


def kernel(x, edge_index, edge_SF_num, W_sd, b_sd, W_ds, b_ds, W6, b6, W5, b5):
    raise NotImplementedError("write your pallas kernel here")


# speedup vs baseline: 27.5104x; 27.5104x over previous
import functools

import jax
import jax.numpy as jnp
from jax import lax
from jax.experimental import pallas as pl
from jax.experimental.pallas import tpu as pltpu
from jax.experimental.pallas import tpu_sc as plsc

ALPHA = 0.5
NC = 2
NS = 16
NW = NC * NS
L = 16
CHUNK = 128
N_DUMMY = 64


def _mesh():
    return plsc.VectorSubcoreMesh(
        core_axis_name="c", subcore_axis_name="s", num_cores=NC, num_subcores=NS
    )


def _make_degree_kernel(n_pad, k_chunks):
    rpt = n_pad // NS

    @functools.partial(
        pl.kernel,
        out_type=jax.ShapeDtypeStruct((NC, 2, n_pad), jnp.float32),
        mesh=_mesh(),
        compiler_params=pltpu.CompilerParams(use_tc_tiling_on_sc=False),
        scratch_types=[
            pltpu.VMEM((k_chunks, CHUNK), jnp.int32),
            pltpu.VMEM((k_chunks, CHUNK), jnp.int32),
            pltpu.VMEM((CHUNK,), jnp.float32),
            pltpu.VMEM((rpt,), jnp.float32),
            pltpu.VMEM_SHARED((n_pad,), jnp.float32),
            pltpu.VMEM_SHARED((n_pad,), jnp.float32),
        ],
    )
    def deg_kernel(src_hbm, dst_hbm, deg_out, src_v, dst_v, ones_v, buf_v,
                   sh_out, sh_in):
        c = lax.axis_index("c")
        s = lax.axis_index("s")
        wid = c * NS + s
        zero16 = jnp.zeros((L,), jnp.float32)
        one16 = jnp.ones((L,), jnp.float32)

        @pl.loop(0, rpt // L)
        def _(j):
            buf_v[pl.ds(j * L, L)] = zero16

        for j in range(CHUNK // L):
            ones_v[pl.ds(j * L, L)] = one16

        sl = pl.ds(s * rpt, rpt)
        pltpu.sync_copy(buf_v, sh_out.at[sl])
        pltpu.sync_copy(buf_v, sh_in.at[sl])
        pltpu.sync_copy(src_hbm.at[wid], src_v)
        pltpu.sync_copy(dst_hbm.at[wid], dst_v)
        plsc.subcore_barrier()
        for k in range(k_chunks):
            pltpu.sync_copy(ones_v, sh_out.at[src_v.at[k]], add=True)
            pltpu.sync_copy(ones_v, sh_in.at[dst_v.at[k]], add=True)
        plsc.subcore_barrier()
        pltpu.sync_copy(sh_out.at[sl], buf_v)
        pltpu.sync_copy(buf_v, deg_out.at[c, 0, sl])
        pltpu.sync_copy(sh_in.at[sl], buf_v)
        pltpu.sync_copy(buf_v, deg_out.at[c, 1, sl])

    return deg_kernel


def _make_agg_kernel(n_pad, k_chunks):
    rpt = n_pad // NS

    @functools.partial(
        pl.kernel,
        out_type=(
            jax.ShapeDtypeStruct((NC, n_pad, L), jnp.float32),
            jax.ShapeDtypeStruct((NC, n_pad, L), jnp.float32),
        ),
        mesh=_mesh(),
        compiler_params=pltpu.CompilerParams(use_tc_tiling_on_sc=False),
        scratch_types=[
            pltpu.VMEM((k_chunks, CHUNK), jnp.int32),
            pltpu.VMEM((k_chunks, CHUNK), jnp.int32),
            pltpu.VMEM((CHUNK, L), jnp.float32),
            pltpu.VMEM((CHUNK, L), jnp.float32),
            pltpu.VMEM((rpt, L), jnp.float32),
            pltpu.VMEM_SHARED((n_pad, L), jnp.float32),
            pltpu.VMEM_SHARED((n_pad, L), jnp.float32),
            pltpu.SemaphoreType.DMA,
            pltpu.SemaphoreType.DMA,
        ],
    )
    def agg_kernel(src_hbm, dst_hbm, pa_hbm, qb_hbm, out_a, out_b,
                   src_v, dst_v, rows_a, rows_b, buf_v, sh_a, sh_b,
                   sem_a, sem_b):
        c = lax.axis_index("c")
        s = lax.axis_index("s")
        wid = c * NS + s
        zero16 = jnp.zeros((L,), jnp.float32)

        @pl.loop(0, rpt)
        def _(j):
            buf_v[j, :] = zero16

        sl = pl.ds(s * rpt, rpt)
        pltpu.sync_copy(buf_v, sh_a.at[sl])
        pltpu.sync_copy(buf_v, sh_b.at[sl])
        pltpu.sync_copy(src_hbm.at[wid], src_v)
        pltpu.sync_copy(dst_hbm.at[wid], dst_v)
        plsc.subcore_barrier()
        for k in range(k_chunks):
            ca = pltpu.async_copy(pa_hbm.at[dst_v.at[k]], rows_a, sem_a)
            cb = pltpu.async_copy(qb_hbm.at[src_v.at[k]], rows_b, sem_b)
            ca.wait()
            pltpu.sync_copy(rows_a, sh_a.at[src_v.at[k]], add=True)
            cb.wait()
            pltpu.sync_copy(rows_b, sh_b.at[dst_v.at[k]], add=True)
        plsc.subcore_barrier()
        pltpu.sync_copy(sh_a.at[sl], buf_v)
        pltpu.sync_copy(buf_v, out_a.at[c, sl])
        pltpu.sync_copy(sh_b.at[sl], buf_v)
        pltpu.sync_copy(buf_v, out_b.at[c, sl])

    return agg_kernel


def _tc_prepare(x_pad, w_sd, w_ds, w6, w5, deg_t):
    n_pad, d = x_pad.shape
    kd = w6.shape[0] + w5.shape[0]
    nblk = 8
    blk = n_pad // nblk

    def body(x_ref, wsd_ref, wds_ref, w6_ref, w5_ref, deg_ref,
             p_ref, q_ref, inv_ref):
        wcat = jnp.concatenate(
            [w6_ref[...], w5_ref[...], jnp.zeros((L - kd, d), jnp.float32)],
            axis=0)
        hi = lax.Precision.HIGHEST
        m_a = jnp.dot(wcat, wsd_ref[...], precision=hi,
                      preferred_element_type=jnp.float32)
        m_at = jnp.dot(wcat, wds_ref[...], precision=hi,
                       preferred_element_type=jnp.float32)
        x = x_ref[...]
        p16 = lax.dot_general(x, m_a, (((1,), (1,)), ((), ())), precision=hi,
                              preferred_element_type=jnp.float32) * ALPHA
        q16 = lax.dot_general(x, m_at, (((1,), (1,)), ((), ())), precision=hi,
                              preferred_element_type=jnp.float32) * (1.0 - ALPHA)
        deg = deg_ref[0] + deg_ref[1]
        inv = jnp.where(deg > 0.0, lax.rsqrt(deg), 0.0)
        inv_ref[...] = inv
        p_ref[...] = p16 * inv[:, 1:2]
        q_ref[...] = q16 * inv[:, 0:1]

    full = lambda *shape: pl.BlockSpec(shape, lambda i: (0,) * len(shape))
    return pl.pallas_call(
        body,
        grid=(nblk,),
        in_specs=[
            pl.BlockSpec((blk, d), lambda i: (i, 0)),
            full(d, d),
            full(d, d),
            full(w6.shape[0], d),
            full(w5.shape[0], d),
            pl.BlockSpec((NC, blk, 2), lambda i: (0, i, 0)),
        ],
        out_specs=(
            pl.BlockSpec((blk, L), lambda i: (i, 0)),
            pl.BlockSpec((blk, L), lambda i: (i, 0)),
            pl.BlockSpec((blk, 2), lambda i: (i, 0)),
        ),
        out_shape=(
            jax.ShapeDtypeStruct((n_pad, L), jnp.float32),
            jax.ShapeDtypeStruct((n_pad, L), jnp.float32),
            jax.ShapeDtypeStruct((n_pad, 2), jnp.float32),
        ),
    )(x_pad, w_sd, w_ds, w6, w5, deg_t)


def _tc_finish(pa, pb, inv, b_sd, b_ds, w6, w5, b6, b5):
    n_pad = pa.shape[1]
    d = b_sd.shape[0]
    kd = w6.shape[0] + w5.shape[0]

    def body(pa_ref, pb_ref, inv_ref, bsd_ref, bds_ref, w6_ref, w5_ref,
             b6_ref, b5_ref, out_ref):
        agg_a = pa_ref[0] + pa_ref[1]
        agg_b = pb_ref[0] + pb_ref[1]
        inv = inv_ref[...]
        wcat = jnp.concatenate(
            [w6_ref[...], w5_ref[...], jnp.zeros((L - kd, d), jnp.float32)],
            axis=0)
        bc = (ALPHA * bsd_ref[...] + (1.0 - ALPHA) * bds_ref[...])[:, None]
        c_col = jnp.dot(wcat, bc, precision=lax.Precision.HIGHEST,
                        preferred_element_type=jnp.float32)
        bcat = jnp.concatenate(
            [b6_ref[...], b5_ref[...], jnp.zeros((L - kd,), jnp.float32)])
        c_row = c_col[:, 0] + bcat
        out_ref[...] = (inv[:, 0:1] * agg_a + inv[:, 1:2] * agg_b
                        + c_row[None, :])

    return pl.pallas_call(
        body,
        out_shape=jax.ShapeDtypeStruct((n_pad, L), jnp.float32),
    )(pa, pb, inv, b_sd, b_ds, w6, w5, b6, b5)


def kernel(x, edge_index, edge_SF_num, W_sd, b_sd, W_ds, b_ds, W6, b6, W5, b5):
    del edge_SF_num
    n, d = x.shape
    e = edge_index.shape[1]

    n_pad = -(-(n + N_DUMMY) // (NS * 8)) * (NS * 8)
    e_pad = -(-e // (NW * CHUNK)) * (NW * CHUNK)
    k_chunks = e_pad // (NW * CHUNK)

    pad_cnt = e_pad - e
    pad_idx = n + (jnp.arange(pad_cnt, dtype=jnp.int32) % N_DUMMY)
    src = jnp.concatenate([edge_index[0], pad_idx])
    dst = jnp.concatenate([edge_index[1], pad_idx])
    src3 = src.reshape(NW, k_chunks, CHUNK)
    dst3 = dst.reshape(NW, k_chunks, CHUNK)

    x_pad = jnp.zeros((n_pad, d), jnp.float32).at[:n].set(x)

    deg_p = _make_degree_kernel(n_pad, k_chunks)(src3, dst3)
    deg_t = jnp.transpose(deg_p, (0, 2, 1))

    p_t, q_t, inv = _tc_prepare(x_pad, W_sd, W_ds, W6, W5, deg_t)

    pa, pb = _make_agg_kernel(n_pad, k_chunks)(src3, dst3, p_t, q_t)

    out = _tc_finish(pa, pb, inv, b_sd, b_ds, W6, W5, b6, b5)
    kd6 = W6.shape[0]
    x_sf = out[:n, :kd6]
    x_ptx = out[:n, kd6:kd6 + W5.shape[0]]
    return (x_sf, x_ptx)
